# baseline (speedup 1.0000x reference)
import functools

import jax
import jax.numpy as jnp
from jax import lax
from jax.experimental import pallas as pl
from jax.experimental.pallas import tpu as pltpu
from jax.experimental.pallas import tpu_sc as plsc

NC = 2
NS = 16
NW = NC * NS
SB = 80
SUB = 5
NP = 10240


def _sc_scatter(E, W, gather):
    epw = E // NW
    nloop = epw // SB
    assert epw % SB == 0
    rpt = NP // NS
    zr = 32
    assert rpt % zr == 0

    mesh = plsc.VectorSubcoreMesh(core_axis_name="c", subcore_axis_name="s")
    scratch = [
        pltpu.VMEM((4, SB), jnp.int32),
        pltpu.VMEM((4, SB, W), jnp.float32),
        pltpu.VMEM((zr, W), jnp.float32),
        pltpu.VMEM_SHARED((NP, W), jnp.float32),
        pltpu.SemaphoreType.DMA,
        pltpu.SemaphoreType.DMA,
    ]
    if gather:
        scratch = [pltpu.VMEM((4, SB), jnp.int32)] + scratch

    def body(*refs):
        if gather:
            (src_h, dst_h, g_hbm, out_hbm,
             s4, d4, rows_v, zb_v, acc, sem_g, sem_s) = refs
        else:
            dst_h, out_hbm, d4, rows_v, zb_v, acc, sem_g, sem_s = refs
        cid = lax.axis_index("c")
        sid = lax.axis_index("s")
        wid = cid * NS + sid
        ebase = wid * epw

        zval = jnp.zeros((16,), jnp.float32)

        def zrow(r, _):
            for c in range(W // 16):
                zb_v[r, pl.ds(c * 16, 16)] = zval
            return 0

        lax.fori_loop(0, zr, zrow, 0)
        if not gather:
            ones = zval + 1.0

            def orow(r, _):
                for c in range(W // 16):
                    rows_v[0, r, pl.ds(c * 16, 16)] = ones
                return 0

            lax.fori_loop(0, SB, orow, 0)

        for j in range(rpt // zr):
            pltpu.sync_copy(zb_v, acc.at[pl.ds(sid * rpt + j * zr, zr)])

        pltpu.sync_copy(dst_h.at[pl.ds(ebase, SB)], d4.at[0])
        if gather:
            pltpu.sync_copy(src_h.at[pl.ds(ebase, SB)], s4.at[0])
        if nloop > 1:
            pltpu.sync_copy(dst_h.at[pl.ds(ebase + SB, SB)], d4.at[1])
            if gather:
                pltpu.sync_copy(src_h.at[pl.ds(ebase + SB, SB)], s4.at[1])
        plsc.subcore_barrier()
        if gather:
            pltpu.async_copy(g_hbm.at[s4.at[0]], rows_v.at[0], sem_g)

        def step(i, _):
            slot = lax.rem(i, 4)
            b1 = lax.rem(i + 1, 4)
            p2 = lax.rem(i + 2, 4)
            rsrc = rows_v.at[slot] if gather else rows_v.at[0]

            @pl.when(i >= 2)
            def _():
                pltpu.make_async_copy(rsrc, acc.at[d4.at[slot]], sem_s).wait()

            @pl.when(i + 2 < nloop)
            def _():
                off = ebase + (i + 2) * SB
                pltpu.sync_copy(dst_h.at[pl.ds(off, SB)], d4.at[p2])
                if gather:
                    pltpu.sync_copy(src_h.at[pl.ds(off, SB)], s4.at[p2])

            if gather:
                @pl.when(i + 1 < nloop)
                def _():
                    pltpu.async_copy(g_hbm.at[s4.at[b1]], rows_v.at[b1], sem_g)

                pltpu.make_async_copy(g_hbm.at[s4.at[slot]], rsrc, sem_g).wait()
            pltpu.async_copy(rsrc, acc.at[d4.at[slot]], sem_s, add=True)
            return 0

        lax.fori_loop(0, nloop, step, 0)
        for _ in range(2):
            pltpu.make_async_copy(rows_v.at[0], acc.at[d4.at[0]], sem_s).wait()
        plsc.subcore_barrier()
        pltpu.sync_copy(acc.at[pl.ds(sid * rpt, rpt)],
                        out_hbm.at[cid, pl.ds(sid * rpt, rpt)])

    out = jax.ShapeDtypeStruct((NC, NP, W), jnp.float32)
    return functools.partial(pl.kernel, body, out_type=out, mesh=mesh,
                             scratch_types=scratch)()


BLK = 1000


def _tc_first(x, W0, degp):
    Nn, D = x.shape
    H = W0.shape[1]

    def body(x_ref, w_ref, p_ref, g_ref, dis_ref):
        p = p_ref[...]
        deg = 1.0 + p[0, :, :1] + p[1, :, :1]
        dis = lax.rsqrt(deg)
        g_ref[...] = jnp.dot(x_ref[...], w_ref[...],
                             preferred_element_type=jnp.float32) * dis
        dis_ref[...] = jnp.broadcast_to(dis, (BLK, 16))

    return pl.pallas_call(
        body,
        grid=(Nn // BLK,),
        in_specs=[
            pl.BlockSpec((BLK, D), lambda i: (i, 0)),
            pl.BlockSpec((D, H), lambda i: (0, 0)),
            pl.BlockSpec((NC, BLK, 128), lambda i: (0, i, 0)),
        ],
        out_specs=[
            pl.BlockSpec((BLK, H), lambda i: (i, 0)),
            pl.BlockSpec((BLK, 16), lambda i: (i, 0)),
        ],
        out_shape=[
            jax.ShapeDtypeStruct((Nn, H), jnp.float32),
            jax.ShapeDtypeStruct((Nn, 16), jnp.float32),
        ],
    )(x, W0, degp)


def _tc_mid(P, g, dis16, b, Wn):
    Nn, H = g.shape
    Ho = Wn.shape[1]

    def body(p_ref, g_ref, d_ref, b_ref, w_ref, o_ref):
        p = p_ref[...]
        dis = d_ref[...][:, :1]
        h = jnp.maximum((p[0] + p[1] + g_ref[...]) * dis + b_ref[...], 0.0)
        o_ref[...] = jnp.dot(h, w_ref[...],
                             preferred_element_type=jnp.float32) * dis

    return pl.pallas_call(
        body,
        grid=(Nn // BLK,),
        in_specs=[
            pl.BlockSpec((NC, BLK, H), lambda i: (0, i, 0)),
            pl.BlockSpec((BLK, H), lambda i: (i, 0)),
            pl.BlockSpec((BLK, 16), lambda i: (i, 0)),
            pl.BlockSpec((1, H), lambda i: (0, 0)),
            pl.BlockSpec((H, Ho), lambda i: (0, 0)),
        ],
        out_specs=pl.BlockSpec((BLK, H), lambda i: (i, 0)),
        out_shape=jax.ShapeDtypeStruct((Nn, Ho), jnp.float32),
    )(P, g, dis16, b, Wn)


def _tc_last_pool_head(P, g, dis16, b, batch2d, G, Wm1, bm1, Wm2, bm2):
    Nn, H = g.shape
    M = Wm1.shape[1]
    nblk = Nn // BLK

    def body(p_ref, g_ref, d_ref, b_ref, t_ref, w1_ref, b1_ref, w2_ref,
             b2_ref, o_ref, s_ref, c_ref):
        i = pl.program_id(0)

        @pl.when(i == 0)
        def _():
            s_ref[...] = jnp.zeros_like(s_ref)
            c_ref[...] = jnp.zeros_like(c_ref)

        p = p_ref[...]
        dis = d_ref[...][:, :1]
        h = jnp.maximum((p[0] + p[1] + g_ref[...]) * dis + b_ref[...], 0.0)
        seg = lax.broadcasted_iota(jnp.int32, (BLK, G), 1)
        oh = (t_ref[...] == seg).astype(jnp.float32)
        dn = (((0,), (0,)), ((), ()))
        s_ref[...] += lax.dot_general(oh, h, dn,
                                      preferred_element_type=jnp.float32)
        c_ref[...] += lax.dot_general(oh, jnp.ones((BLK, H), jnp.float32), dn,
                                      preferred_element_type=jnp.float32)

        @pl.when(i == nblk - 1)
        def _():
            pooled = s_ref[...] / jnp.maximum(c_ref[...], 1.0)
            hm = jnp.maximum(jnp.dot(pooled, w1_ref[...],
                                     preferred_element_type=jnp.float32)
                             + b1_ref[...], 0.0)
            o_ref[...] = jnp.dot(hm, w2_ref[...],
                                 preferred_element_type=jnp.float32) + b2_ref[...]

    return pl.pallas_call(
        body,
        grid=(nblk,),
        in_specs=[
            pl.BlockSpec((NC, BLK, H), lambda i: (0, i, 0)),
            pl.BlockSpec((BLK, H), lambda i: (i, 0)),
            pl.BlockSpec((BLK, 16), lambda i: (i, 0)),
            pl.BlockSpec((1, H), lambda i: (0, 0)),
            pl.BlockSpec((BLK, 1), lambda i: (i, 0)),
            pl.BlockSpec((H, M), lambda i: (0, 0)),
            pl.BlockSpec((1, M), lambda i: (0, 0)),
            pl.BlockSpec((M, 1), lambda i: (0, 0)),
            pl.BlockSpec((1, 1), lambda i: (0, 0)),
        ],
        out_specs=pl.BlockSpec((G, 1), lambda i: (0, 0)),
        out_shape=jax.ShapeDtypeStruct((G, 1), jnp.float32),
        scratch_shapes=[
            pltpu.VMEM((G, H), jnp.float32),
            pltpu.VMEM((G, H), jnp.float32),
        ],
    )(P, g, dis16, b, batch2d, Wm1, bm1, Wm2, bm2)


def kernel(x, edge_index, batch, W0, b0, W1, b1, W2, b2, Wm1, bm1, Wm2, bm2):
    Nn, D = x.shape
    E = edge_index.shape[1]
    G = 64
    src = edge_index[0]
    dst = edge_index[1]

    degp = _sc_scatter(E, 128, gather=False)(dst)
    g0, dis16 = _tc_first(x, W0, degp)
    P = _sc_scatter(E, 128, gather=True)(src, dst, g0)
    g1 = _tc_mid(P, g0, dis16, b0.reshape(1, -1), W1)
    P = _sc_scatter(E, 128, gather=True)(src, dst, g1)
    g2 = _tc_mid(P, g1, dis16, b1.reshape(1, -1), W2)
    P = _sc_scatter(E, 128, gather=True)(src, dst, g2)
    out = _tc_last_pool_head(P, g2, dis16, b2.reshape(1, -1),
                             batch.reshape(-1, 1), G,
                             Wm1, bm1.reshape(1, -1), Wm2, bm2.reshape(1, -1))
    return out.reshape(-1)


# speedup vs baseline: 1.0634x; 1.0634x over previous
import functools

import jax
import jax.numpy as jnp
from jax import lax
from jax.experimental import pallas as pl
from jax.experimental.pallas import tpu as pltpu
from jax.experimental.pallas import tpu_sc as plsc

NC = 2
NS = 16
NW = NC * NS
SB = 80
SUB = 5
NP = 10240


def _sc_scatter(E, W, gather):
    epw = E // NW
    nloop = epw // SB
    assert epw % SB == 0
    rpt = NP // NS
    zr = 32
    assert rpt % zr == 0

    mesh = plsc.VectorSubcoreMesh(core_axis_name="c", subcore_axis_name="s")
    scratch = [
        pltpu.VMEM((6, SB), jnp.int32),
        pltpu.VMEM((4, SB, W), jnp.float32),
        pltpu.VMEM((zr, W), jnp.float32),
        pltpu.VMEM_SHARED((NP, W), jnp.float32),
        pltpu.SemaphoreType.DMA,
        pltpu.SemaphoreType.DMA,
    ]
    if gather:
        scratch = [pltpu.VMEM((6, SB), jnp.int32)] + scratch

    def body(*refs):
        if gather:
            (src_h, dst_h, g_hbm, out_hbm,
             s4, d4, rows_v, zb_v, acc, sem_g, sem_s) = refs
        else:
            dst_h, out_hbm, d4, rows_v, zb_v, acc, sem_g, sem_s = refs
        cid = lax.axis_index("c")
        sid = lax.axis_index("s")
        wid = cid * NS + sid
        ebase = wid * epw

        zval = jnp.zeros((16,), jnp.float32)

        def zrow(r, _):
            for c in range(W // 16):
                zb_v[r, pl.ds(c * 16, 16)] = zval
            return 0

        lax.fori_loop(0, zr, zrow, 0)
        if not gather:
            ones = zval + 1.0

            def orow(r, _):
                for c in range(W // 16):
                    rows_v[0, r, pl.ds(c * 16, 16)] = ones
                return 0

            lax.fori_loop(0, SB, orow, 0)

        for j in range(rpt // zr):
            pltpu.sync_copy(zb_v, acc.at[pl.ds(sid * rpt + j * zr, zr)])

        for j in range(3):
            if nloop > j:
                pltpu.sync_copy(dst_h.at[pl.ds(ebase + j * SB, SB)], d4.at[j])
                if gather:
                    pltpu.sync_copy(src_h.at[pl.ds(ebase + j * SB, SB)],
                                    s4.at[j])
        plsc.subcore_barrier()
        if gather:
            pltpu.async_copy(g_hbm.at[s4.at[0]], rows_v.at[0], sem_g)
            if nloop > 1:
                pltpu.async_copy(g_hbm.at[s4.at[1]], rows_v.at[1], sem_g)

        def step(i, _):
            slot = lax.rem(i, 4)
            islot = lax.rem(i, 6)
            b2 = lax.rem(i + 2, 4)
            i2 = lax.rem(i + 2, 6)
            p3 = lax.rem(i + 3, 6)
            rsrc = rows_v.at[slot] if gather else rows_v.at[0]

            @pl.when(i >= 2)
            def _():
                pltpu.make_async_copy(rsrc, acc.at[d4.at[islot]], sem_s).wait()

            @pl.when(i + 3 < nloop)
            def _():
                off = ebase + (i + 3) * SB
                pltpu.sync_copy(dst_h.at[pl.ds(off, SB)], d4.at[p3])
                if gather:
                    pltpu.sync_copy(src_h.at[pl.ds(off, SB)], s4.at[p3])

            if gather:
                @pl.when(i + 2 < nloop)
                def _():
                    pltpu.async_copy(g_hbm.at[s4.at[i2]], rows_v.at[b2], sem_g)

                pltpu.make_async_copy(g_hbm.at[s4.at[islot]], rsrc,
                                      sem_g).wait()
            pltpu.async_copy(rsrc, acc.at[d4.at[islot]], sem_s, add=True)
            return 0

        lax.fori_loop(0, nloop, step, 0)
        for _ in range(2):
            pltpu.make_async_copy(rows_v.at[0], acc.at[d4.at[0]], sem_s).wait()
        plsc.subcore_barrier()
        pltpu.sync_copy(acc.at[pl.ds(sid * rpt, rpt)],
                        out_hbm.at[cid, pl.ds(sid * rpt, rpt)])

    out = jax.ShapeDtypeStruct((NC, NP, W), jnp.float32)
    return functools.partial(pl.kernel, body, out_type=out, mesh=mesh,
                             scratch_types=scratch)()


BLK = 1000


def _tc_first(x, W0, degp):
    Nn, D = x.shape
    H = W0.shape[1]

    def body(x_ref, w_ref, p_ref, g_ref, dis_ref):
        p = p_ref[...]
        deg = 1.0 + p[0, :, :1] + p[1, :, :1]
        dis = lax.rsqrt(deg)
        g_ref[...] = jnp.dot(x_ref[...], w_ref[...],
                             preferred_element_type=jnp.float32) * dis
        dis_ref[...] = jnp.broadcast_to(dis, (BLK, 16))

    return pl.pallas_call(
        body,
        grid=(Nn // BLK,),
        in_specs=[
            pl.BlockSpec((BLK, D), lambda i: (i, 0)),
            pl.BlockSpec((D, H), lambda i: (0, 0)),
            pl.BlockSpec((NC, BLK, 128), lambda i: (0, i, 0)),
        ],
        out_specs=[
            pl.BlockSpec((BLK, H), lambda i: (i, 0)),
            pl.BlockSpec((BLK, 16), lambda i: (i, 0)),
        ],
        out_shape=[
            jax.ShapeDtypeStruct((Nn, H), jnp.float32),
            jax.ShapeDtypeStruct((Nn, 16), jnp.float32),
        ],
    )(x, W0, degp)


def _tc_mid(P, g, dis16, b, Wn):
    Nn, H = g.shape
    Ho = Wn.shape[1]

    def body(p_ref, g_ref, d_ref, b_ref, w_ref, o_ref):
        p = p_ref[...]
        dis = d_ref[...][:, :1]
        h = jnp.maximum((p[0] + p[1] + g_ref[...]) * dis + b_ref[...], 0.0)
        o_ref[...] = jnp.dot(h, w_ref[...],
                             preferred_element_type=jnp.float32) * dis

    return pl.pallas_call(
        body,
        grid=(Nn // BLK,),
        in_specs=[
            pl.BlockSpec((NC, BLK, H), lambda i: (0, i, 0)),
            pl.BlockSpec((BLK, H), lambda i: (i, 0)),
            pl.BlockSpec((BLK, 16), lambda i: (i, 0)),
            pl.BlockSpec((1, H), lambda i: (0, 0)),
            pl.BlockSpec((H, Ho), lambda i: (0, 0)),
        ],
        out_specs=pl.BlockSpec((BLK, H), lambda i: (i, 0)),
        out_shape=jax.ShapeDtypeStruct((Nn, Ho), jnp.float32),
    )(P, g, dis16, b, Wn)


def _tc_last_pool_head(P, g, dis16, b, batch2d, G, Wm1, bm1, Wm2, bm2):
    Nn, H = g.shape
    M = Wm1.shape[1]
    nblk = Nn // BLK

    def body(p_ref, g_ref, d_ref, b_ref, t_ref, w1_ref, b1_ref, w2_ref,
             b2_ref, o_ref, s_ref, c_ref):
        i = pl.program_id(0)

        @pl.when(i == 0)
        def _():
            s_ref[...] = jnp.zeros_like(s_ref)
            c_ref[...] = jnp.zeros_like(c_ref)

        p = p_ref[...]
        dis = d_ref[...][:, :1]
        h = jnp.maximum((p[0] + p[1] + g_ref[...]) * dis + b_ref[...], 0.0)
        seg = lax.broadcasted_iota(jnp.int32, (BLK, G), 1)
        oh = (t_ref[...] == seg).astype(jnp.float32)
        dn = (((0,), (0,)), ((), ()))
        s_ref[...] += lax.dot_general(oh, h, dn,
                                      preferred_element_type=jnp.float32)
        c_ref[...] += lax.dot_general(oh, jnp.ones((BLK, H), jnp.float32), dn,
                                      preferred_element_type=jnp.float32)

        @pl.when(i == nblk - 1)
        def _():
            pooled = s_ref[...] / jnp.maximum(c_ref[...], 1.0)
            hm = jnp.maximum(jnp.dot(pooled, w1_ref[...],
                                     preferred_element_type=jnp.float32)
                             + b1_ref[...], 0.0)
            o_ref[...] = jnp.dot(hm, w2_ref[...],
                                 preferred_element_type=jnp.float32) + b2_ref[...]

    return pl.pallas_call(
        body,
        grid=(nblk,),
        in_specs=[
            pl.BlockSpec((NC, BLK, H), lambda i: (0, i, 0)),
            pl.BlockSpec((BLK, H), lambda i: (i, 0)),
            pl.BlockSpec((BLK, 16), lambda i: (i, 0)),
            pl.BlockSpec((1, H), lambda i: (0, 0)),
            pl.BlockSpec((BLK, 1), lambda i: (i, 0)),
            pl.BlockSpec((H, M), lambda i: (0, 0)),
            pl.BlockSpec((1, M), lambda i: (0, 0)),
            pl.BlockSpec((M, 1), lambda i: (0, 0)),
            pl.BlockSpec((1, 1), lambda i: (0, 0)),
        ],
        out_specs=pl.BlockSpec((G, 1), lambda i: (0, 0)),
        out_shape=jax.ShapeDtypeStruct((G, 1), jnp.float32),
        scratch_shapes=[
            pltpu.VMEM((G, H), jnp.float32),
            pltpu.VMEM((G, H), jnp.float32),
        ],
    )(P, g, dis16, b, batch2d, Wm1, bm1, Wm2, bm2)


def kernel(x, edge_index, batch, W0, b0, W1, b1, W2, b2, Wm1, bm1, Wm2, bm2):
    Nn, D = x.shape
    E = edge_index.shape[1]
    G = 64
    src = edge_index[0]
    dst = edge_index[1]

    degp = _sc_scatter(E, 128, gather=False)(dst)
    g0, dis16 = _tc_first(x, W0, degp)
    P = _sc_scatter(E, 128, gather=True)(src, dst, g0)
    g1 = _tc_mid(P, g0, dis16, b0.reshape(1, -1), W1)
    P = _sc_scatter(E, 128, gather=True)(src, dst, g1)
    g2 = _tc_mid(P, g1, dis16, b1.reshape(1, -1), W2)
    P = _sc_scatter(E, 128, gather=True)(src, dst, g2)
    out = _tc_last_pool_head(P, g2, dis16, b2.reshape(1, -1),
                             batch.reshape(-1, 1), G,
                             Wm1, bm1.reshape(1, -1), Wm2, bm2.reshape(1, -1))
    return out.reshape(-1)
